# baseline (speedup 1.0000x reference)
import functools

import jax
import jax.numpy as jnp
from jax.experimental import pallas as pl
from jax.experimental.pallas import tpu as pltpu

ALPHA = 0.25
GAMMA = 2.0
DIVIDE_LINE = 1.0 / 9.0

LANE = 128
ROWS = 784
RB = 16
CR = 16


def _match_kernel(boxes_ref, labels_ref, anc_ref, reg_ref,
                  posf_ref, ignf_ref, labf_ref, rsum_ref, npos_ref,
                  *, n_anchors, n_boxes):
    j = pl.program_id(1)

    @pl.when(j == 0)
    def _init():
        rsum_ref[...] = jnp.zeros_like(rsum_ref)
        npos_ref[...] = jnp.zeros_like(npos_ref)

    ax0 = anc_ref[0]
    ay0 = anc_ref[1]
    ax1 = anc_ref[2]
    ay1 = anc_ref[3]
    area_a = (ax1 - ax0) * (ay1 - ay0)

    best_in = jnp.full(ax0.shape, -1.0, jnp.float32)
    best_un = jnp.ones(ax0.shape, jnp.float32)
    best_lab = jnp.zeros(ax0.shape, jnp.float32)
    bgx0 = jnp.zeros(ax0.shape, jnp.float32)
    bgy0 = jnp.zeros(ax0.shape, jnp.float32)
    bgx1 = jnp.zeros(ax0.shape, jnp.float32)
    bgy1 = jnp.zeros(ax0.shape, jnp.float32)

    for m in range(n_boxes):
        bx0 = boxes_ref[0, 0, 4 * m + 0]
        by0 = boxes_ref[0, 0, 4 * m + 1]
        bx1 = boxes_ref[0, 0, 4 * m + 2]
        by1 = boxes_ref[0, 0, 4 * m + 3]
        area_b = (bx1 - bx0) * (by1 - by0)
        wx = jnp.clip(jnp.minimum(ax1, bx1) - jnp.maximum(ax0, bx0), 0.0)
        wy = jnp.clip(jnp.minimum(ay1, by1) - jnp.maximum(ay0, by0), 0.0)
        inter = wx * wy
        union = jnp.maximum(area_a + area_b - inter, 1e-09)
        upd = inter * best_un > best_in * union
        best_in = jnp.where(upd, inter, best_in)
        best_un = jnp.where(upd, union, best_un)
        lab = labels_ref[0, 0, m]
        best_lab = jnp.where(upd, lab, best_lab)
        bgx0 = jnp.where(upd, bx0, bgx0)
        bgy0 = jnp.where(upd, by0, bgy0)
        bgx1 = jnp.where(upd, bx1, bgx1)
        bgy1 = jnp.where(upd, by1, bgy1)
    best_iou = best_in / best_un

    a_idx = (jax.lax.broadcasted_iota(jnp.int32, ax0.shape, 0) * LANE
             + jax.lax.broadcasted_iota(jnp.int32, ax0.shape, 1)
             + j * (RB * LANE))
    validf = (a_idx < n_anchors).astype(jnp.float32)

    posf = (best_iou >= 0.5).astype(jnp.float32) * validf
    ignf = (jnp.logical_and(best_iou >= 0.4, best_iou < 0.5)
            .astype(jnp.float32) * validf)
    posf_ref[0] = posf
    ignf_ref[0] = ignf
    labf_ref[0] = best_lab
    npos_ref[0] += jnp.sum(posf, axis=0, keepdims=True)

    aw = ax1 - ax0
    ah = ay1 - ay0
    acx = ax0 + 0.5 * aw
    acy = ay0 + 0.5 * ah
    gw = bgx1 - bgx0
    gh = bgy1 - bgy0
    gcx = bgx0 + 0.5 * gw
    gcy = bgy0 + 0.5 * gh
    t0 = (gcx - acx) / aw
    t1 = (gcy - acy) / ah
    t2 = jnp.log(gw / aw)
    t3 = jnp.log(gh / ah)

    sl_acc = jnp.zeros(ax0.shape, jnp.float32)
    for k, t in enumerate((t0, t1, t2, t3)):
        diff = jnp.abs(reg_ref[0, k] - t)
        sl_acc += jnp.where(diff < DIVIDE_LINE,
                            0.5 / DIVIDE_LINE * diff * diff,
                            diff - 0.5 * DIVIDE_LINE)
    rsum_ref[0] += jnp.sum(sl_acc * posf, axis=0, keepdims=True)


def _dense_kernel(cls_ref, labi_ref, posf_ref, ignf_ref,
                  bsum_ref, corr_ref, pam_ref, *, nb):
    j = pl.program_id(0)

    @pl.when(j % nb == 0)
    def _init():
        bsum_ref[...] = jnp.zeros_like(bsum_ref)
        pam_ref[...] = jnp.zeros_like(pam_ref)

    p = jnp.clip(cls_ref[...], 1e-06, 1.0 - 1e-06)
    bsum_ref[0] += jnp.full((1, LANE), jnp.sum(jnp.log(1.0 - p) * p * p))

    labm = labi_ref[0] - (j % nb) * CR
    at_m = labm == jax.lax.broadcasted_iota(jnp.int32, p.shape, 0)
    pam_ref[...] += jnp.sum(jnp.where(at_m, p, 0.0), axis=0, keepdims=True)

    @pl.when(j % nb == nb - 1)
    def _corr():
        pm = pam_ref[...]
        qm = 1.0 - pm
        bg_m = (1.0 - ALPHA) * -jnp.log(qm) * pm * pm
        fg_m = ALPHA * -jnp.log(pm) * qm * qm
        corr = (posf_ref[0] * (fg_m - bg_m) - ignf_ref[0] * bg_m)
        corr_ref[0] = jnp.full((1, LANE), jnp.sum(corr))


@jax.jit
def kernel(classifications, regressions, anchors, gt_boxes, gt_labels):
    n, a, c = classifications.shape
    m = gt_boxes.shape[1]
    a_pad = ROWS * LANE

    anc_pad = jnp.concatenate(
        [anchors,
         jnp.broadcast_to(jnp.array([0.0, 0.0, 1.0, 1.0], jnp.float32),
                          (a_pad - a, 4))], axis=0)
    anc_t = anc_pad.T.reshape(4, ROWS, LANE)
    reg_pad = jnp.pad(regressions, ((0, 0), (0, a_pad - a), (0, 0)))
    reg_t = reg_pad.transpose(0, 2, 1).reshape(n, 4, ROWS, LANE)
    boxes_s = gt_boxes.reshape(n, 1, 4 * m)
    labels_s = gt_labels.astype(jnp.float32).reshape(n, 1, m)

    posf, ignf, labf, rsum, npos = pl.pallas_call(
        functools.partial(_match_kernel, n_anchors=a, n_boxes=m),
        grid=(n, ROWS // RB),
        in_specs=[
            pl.BlockSpec((1, 1, 4 * m), lambda i, j: (i, 0, 0),
                         memory_space=pltpu.MemorySpace.SMEM),
            pl.BlockSpec((1, 1, m), lambda i, j: (i, 0, 0),
                         memory_space=pltpu.MemorySpace.SMEM),
            pl.BlockSpec((4, RB, LANE), lambda i, j: (0, j, 0)),
            pl.BlockSpec((1, 4, RB, LANE), lambda i, j: (i, 0, j, 0)),
        ],
        out_specs=[
            pl.BlockSpec((1, RB, LANE), lambda i, j: (i, j, 0)),
            pl.BlockSpec((1, RB, LANE), lambda i, j: (i, j, 0)),
            pl.BlockSpec((1, RB, LANE), lambda i, j: (i, j, 0)),
            pl.BlockSpec((1, 1, LANE), lambda i, j: (i, 0, 0)),
            pl.BlockSpec((1, 1, LANE), lambda i, j: (i, 0, 0)),
        ],
        out_shape=[
            jax.ShapeDtypeStruct((n, ROWS, LANE), jnp.float32),
            jax.ShapeDtypeStruct((n, ROWS, LANE), jnp.float32),
            jax.ShapeDtypeStruct((n, ROWS, LANE), jnp.float32),
            jax.ShapeDtypeStruct((n, 1, LANE), jnp.float32),
            jax.ShapeDtypeStruct((n, 1, LANE), jnp.float32),
        ],
    )(boxes_s, labels_s, anc_t, reg_t)

    labv = labf.reshape(n, a_pad)[:, :a].reshape(n, 1, a).astype(jnp.int32)
    posv = posf.reshape(n, a_pad)[:, :a].reshape(n, 1, a)
    ignv = ignf.reshape(n, a_pad)[:, :a].reshape(n, 1, a)

    cls2d = classifications.transpose(0, 2, 1).reshape(n * c, a)
    nb = c // CR
    bsum_d, corr_d = pl.pallas_call(
        functools.partial(_dense_kernel, nb=nb),
        grid=(n * nb,),
        in_specs=[
            pl.BlockSpec((CR, a), lambda j: (j, 0)),
            pl.BlockSpec((1, 1, a), lambda j: (j // nb, 0, 0)),
            pl.BlockSpec((1, 1, a), lambda j: (j // nb, 0, 0)),
            pl.BlockSpec((1, 1, a), lambda j: (j // nb, 0, 0)),
        ],
        out_specs=[
            pl.BlockSpec((1, 1, LANE), lambda j: (j // nb, 0, 0)),
            pl.BlockSpec((1, 1, LANE), lambda j: (j // nb, 0, 0)),
        ],
        out_shape=[
            jax.ShapeDtypeStruct((n, 1, LANE), jnp.float32),
            jax.ShapeDtypeStruct((n, 1, LANE), jnp.float32),
        ],
        scratch_shapes=[pltpu.VMEM((1, a), jnp.float32)],
    )(cls2d, labv, posv, ignv)

    csum = -(1.0 - ALPHA) * bsum_d[:, 0, 0] + corr_d[:, 0, 0]
    np_ = jnp.sum(npos, axis=(1, 2))
    rs = jnp.sum(rsum, axis=(1, 2))
    denom = jnp.maximum(np_, 1.0)
    class_loss = jnp.mean(csum / denom)
    reg_loss = jnp.mean(jnp.where(np_ > 0, rs / (denom * 4.0), 0.0))
    return (class_loss, reg_loss)


# speedup vs baseline: 1.3469x; 1.3469x over previous
import functools

import jax
import jax.numpy as jnp
from jax.experimental import pallas as pl
from jax.experimental.pallas import tpu as pltpu

ALPHA = 0.25
GAMMA = 2.0
DIVIDE_LINE = 1.0 / 9.0

LANE = 128
ROWS = 784
RB = 56
CR = 16


def _match_kernel(boxes_ref, labels_ref, anc_ref, reg_ref,
                  posf_ref, ignf_ref, labf_ref, rsum_ref, npos_ref,
                  *, n_anchors, n_boxes):
    j = pl.program_id(1)

    @pl.when(j == 0)
    def _init():
        rsum_ref[...] = jnp.zeros_like(rsum_ref)
        npos_ref[...] = jnp.zeros_like(npos_ref)

    ax0 = anc_ref[0]
    ay0 = anc_ref[1]
    ax1 = anc_ref[2]
    ay1 = anc_ref[3]
    area_a = (ax1 - ax0) * (ay1 - ay0)

    best_in = jnp.full(ax0.shape, -1.0, jnp.float32)
    best_un = jnp.ones(ax0.shape, jnp.float32)
    best_lab = jnp.zeros(ax0.shape, jnp.float32)
    bgx0 = jnp.zeros(ax0.shape, jnp.float32)
    bgy0 = jnp.zeros(ax0.shape, jnp.float32)
    bgx1 = jnp.zeros(ax0.shape, jnp.float32)
    bgy1 = jnp.zeros(ax0.shape, jnp.float32)

    for m in range(n_boxes):
        bx0 = boxes_ref[0, 0, 4 * m + 0]
        by0 = boxes_ref[0, 0, 4 * m + 1]
        bx1 = boxes_ref[0, 0, 4 * m + 2]
        by1 = boxes_ref[0, 0, 4 * m + 3]
        area_b = (bx1 - bx0) * (by1 - by0)
        wx = jnp.clip(jnp.minimum(ax1, bx1) - jnp.maximum(ax0, bx0), 0.0)
        wy = jnp.clip(jnp.minimum(ay1, by1) - jnp.maximum(ay0, by0), 0.0)
        inter = wx * wy
        union = jnp.maximum(area_a + area_b - inter, 1e-09)
        upd = inter * best_un > best_in * union
        best_in = jnp.where(upd, inter, best_in)
        best_un = jnp.where(upd, union, best_un)
        lab = labels_ref[0, 0, m]
        best_lab = jnp.where(upd, lab, best_lab)
        bgx0 = jnp.where(upd, bx0, bgx0)
        bgy0 = jnp.where(upd, by0, bgy0)
        bgx1 = jnp.where(upd, bx1, bgx1)
        bgy1 = jnp.where(upd, by1, bgy1)
    best_iou = best_in / best_un

    a_idx = (jax.lax.broadcasted_iota(jnp.int32, ax0.shape, 0) * LANE
             + jax.lax.broadcasted_iota(jnp.int32, ax0.shape, 1)
             + j * (RB * LANE))
    validf = (a_idx < n_anchors).astype(jnp.float32)

    posf = (best_iou >= 0.5).astype(jnp.float32) * validf
    ignf = (jnp.logical_and(best_iou >= 0.4, best_iou < 0.5)
            .astype(jnp.float32) * validf)
    posf_ref[0] = posf
    ignf_ref[0] = ignf
    labf_ref[0] = best_lab
    npos_ref[0] += jnp.sum(posf, axis=0, keepdims=True)

    aw = ax1 - ax0
    ah = ay1 - ay0
    acx = ax0 + 0.5 * aw
    acy = ay0 + 0.5 * ah
    gw = bgx1 - bgx0
    gh = bgy1 - bgy0
    gcx = bgx0 + 0.5 * gw
    gcy = bgy0 + 0.5 * gh
    t0 = (gcx - acx) / aw
    t1 = (gcy - acy) / ah
    t2 = jnp.log(gw / aw)
    t3 = jnp.log(gh / ah)

    sl_acc = jnp.zeros(ax0.shape, jnp.float32)
    for k, t in enumerate((t0, t1, t2, t3)):
        diff = jnp.abs(reg_ref[0, k] - t)
        sl_acc += jnp.where(diff < DIVIDE_LINE,
                            0.5 / DIVIDE_LINE * diff * diff,
                            diff - 0.5 * DIVIDE_LINE)
    rsum_ref[0] += jnp.sum(sl_acc * posf, axis=0, keepdims=True)


def _dense_kernel(cls_ref, labi_ref, posf_ref, ignf_ref,
                  bsum_ref, corr_ref, pam_ref, *, nb):
    j = pl.program_id(0)

    @pl.when(j % nb == 0)
    def _init():
        bsum_ref[...] = jnp.zeros_like(bsum_ref)
        pam_ref[...] = jnp.zeros_like(pam_ref)

    p = jnp.clip(cls_ref[...], 1e-06, 1.0 - 1e-06)
    bsum_ref[0] += jnp.full((1, LANE), jnp.sum(jnp.log(1.0 - p) * p * p))

    labm = labi_ref[0] - (j % nb) * CR
    at_m = labm == jax.lax.broadcasted_iota(jnp.int32, p.shape, 0)
    pam_ref[...] += jnp.sum(jnp.where(at_m, p, 0.0), axis=0, keepdims=True)

    @pl.when(j % nb == nb - 1)
    def _corr():
        pm = pam_ref[...]
        qm = 1.0 - pm
        bg_m = (1.0 - ALPHA) * -jnp.log(qm) * pm * pm
        fg_m = ALPHA * -jnp.log(pm) * qm * qm
        corr = (posf_ref[0] * (fg_m - bg_m) - ignf_ref[0] * bg_m)
        corr_ref[0] = jnp.full((1, LANE), jnp.sum(corr))


@jax.jit
def kernel(classifications, regressions, anchors, gt_boxes, gt_labels):
    n, a, c = classifications.shape
    m = gt_boxes.shape[1]
    a_pad = ROWS * LANE

    anc_pad = jnp.concatenate(
        [anchors,
         jnp.broadcast_to(jnp.array([0.0, 0.0, 1.0, 1.0], jnp.float32),
                          (a_pad - a, 4))], axis=0)
    anc_t = anc_pad.T.reshape(4, ROWS, LANE)
    reg_pad = jnp.pad(regressions, ((0, 0), (0, a_pad - a), (0, 0)))
    reg_t = reg_pad.transpose(0, 2, 1).reshape(n, 4, ROWS, LANE)
    boxes_s = gt_boxes.reshape(n, 1, 4 * m)
    labels_s = gt_labels.astype(jnp.float32).reshape(n, 1, m)

    posf, ignf, labf, rsum, npos = pl.pallas_call(
        functools.partial(_match_kernel, n_anchors=a, n_boxes=m),
        grid=(n, ROWS // RB),
        in_specs=[
            pl.BlockSpec((1, 1, 4 * m), lambda i, j: (i, 0, 0),
                         memory_space=pltpu.MemorySpace.SMEM),
            pl.BlockSpec((1, 1, m), lambda i, j: (i, 0, 0),
                         memory_space=pltpu.MemorySpace.SMEM),
            pl.BlockSpec((4, RB, LANE), lambda i, j: (0, j, 0)),
            pl.BlockSpec((1, 4, RB, LANE), lambda i, j: (i, 0, j, 0)),
        ],
        out_specs=[
            pl.BlockSpec((1, RB, LANE), lambda i, j: (i, j, 0)),
            pl.BlockSpec((1, RB, LANE), lambda i, j: (i, j, 0)),
            pl.BlockSpec((1, RB, LANE), lambda i, j: (i, j, 0)),
            pl.BlockSpec((1, 1, LANE), lambda i, j: (i, 0, 0)),
            pl.BlockSpec((1, 1, LANE), lambda i, j: (i, 0, 0)),
        ],
        out_shape=[
            jax.ShapeDtypeStruct((n, ROWS, LANE), jnp.float32),
            jax.ShapeDtypeStruct((n, ROWS, LANE), jnp.float32),
            jax.ShapeDtypeStruct((n, ROWS, LANE), jnp.float32),
            jax.ShapeDtypeStruct((n, 1, LANE), jnp.float32),
            jax.ShapeDtypeStruct((n, 1, LANE), jnp.float32),
        ],
    )(boxes_s, labels_s, anc_t, reg_t)

    labv = labf.reshape(n, a_pad)[:, :a].reshape(n, 1, a).astype(jnp.int32)
    posv = posf.reshape(n, a_pad)[:, :a].reshape(n, 1, a)
    ignv = ignf.reshape(n, a_pad)[:, :a].reshape(n, 1, a)

    cls2d = classifications.transpose(0, 2, 1).reshape(n * c, a)
    nb = c // CR
    bsum_d, corr_d = pl.pallas_call(
        functools.partial(_dense_kernel, nb=nb),
        grid=(n * nb,),
        in_specs=[
            pl.BlockSpec((CR, a), lambda j: (j, 0)),
            pl.BlockSpec((1, 1, a), lambda j: (j // nb, 0, 0)),
            pl.BlockSpec((1, 1, a), lambda j: (j // nb, 0, 0)),
            pl.BlockSpec((1, 1, a), lambda j: (j // nb, 0, 0)),
        ],
        out_specs=[
            pl.BlockSpec((1, 1, LANE), lambda j: (j // nb, 0, 0)),
            pl.BlockSpec((1, 1, LANE), lambda j: (j // nb, 0, 0)),
        ],
        out_shape=[
            jax.ShapeDtypeStruct((n, 1, LANE), jnp.float32),
            jax.ShapeDtypeStruct((n, 1, LANE), jnp.float32),
        ],
        scratch_shapes=[pltpu.VMEM((1, a), jnp.float32)],
    )(cls2d, labv, posv, ignv)

    csum = -(1.0 - ALPHA) * bsum_d[:, 0, 0] + corr_d[:, 0, 0]
    np_ = jnp.sum(npos, axis=(1, 2))
    rs = jnp.sum(rsum, axis=(1, 2))
    denom = jnp.maximum(np_, 1.0)
    class_loss = jnp.mean(csum / denom)
    reg_loss = jnp.mean(jnp.where(np_ > 0, rs / (denom * 4.0), 0.0))
    return (class_loss, reg_loss)


# speedup vs baseline: 1.3861x; 1.0291x over previous
import functools

import jax
import jax.numpy as jnp
from jax.experimental import pallas as pl
from jax.experimental.pallas import tpu as pltpu

ALPHA = 0.25
GAMMA = 2.0
DIVIDE_LINE = 1.0 / 9.0

LANE = 128
ROWS = 784
RB = 112
CR = 16


def _match_kernel(boxes_ref, labels_ref, anc_ref, reg_ref,
                  posf_ref, ignf_ref, labf_ref, rsum_ref, npos_ref,
                  *, n_anchors, n_boxes):
    j = pl.program_id(1)

    @pl.when(j == 0)
    def _init():
        rsum_ref[...] = jnp.zeros_like(rsum_ref)
        npos_ref[...] = jnp.zeros_like(npos_ref)

    ax0 = anc_ref[0]
    ay0 = anc_ref[1]
    ax1 = anc_ref[2]
    ay1 = anc_ref[3]
    area_a = (ax1 - ax0) * (ay1 - ay0)

    best_in = jnp.full(ax0.shape, -1.0, jnp.float32)
    best_un = jnp.ones(ax0.shape, jnp.float32)
    best_lab = jnp.zeros(ax0.shape, jnp.float32)
    bgx0 = jnp.zeros(ax0.shape, jnp.float32)
    bgy0 = jnp.zeros(ax0.shape, jnp.float32)
    bgx1 = jnp.zeros(ax0.shape, jnp.float32)
    bgy1 = jnp.zeros(ax0.shape, jnp.float32)

    for m in range(n_boxes):
        bx0 = boxes_ref[0, 0, 4 * m + 0]
        by0 = boxes_ref[0, 0, 4 * m + 1]
        bx1 = boxes_ref[0, 0, 4 * m + 2]
        by1 = boxes_ref[0, 0, 4 * m + 3]
        area_b = (bx1 - bx0) * (by1 - by0)
        wx = jnp.clip(jnp.minimum(ax1, bx1) - jnp.maximum(ax0, bx0), 0.0)
        wy = jnp.clip(jnp.minimum(ay1, by1) - jnp.maximum(ay0, by0), 0.0)
        inter = wx * wy
        union = jnp.maximum(area_a + area_b - inter, 1e-09)
        upd = inter * best_un > best_in * union
        best_in = jnp.where(upd, inter, best_in)
        best_un = jnp.where(upd, union, best_un)
        lab = labels_ref[0, 0, m]
        best_lab = jnp.where(upd, lab, best_lab)
        bgx0 = jnp.where(upd, bx0, bgx0)
        bgy0 = jnp.where(upd, by0, bgy0)
        bgx1 = jnp.where(upd, bx1, bgx1)
        bgy1 = jnp.where(upd, by1, bgy1)
    best_iou = best_in / best_un

    a_idx = (jax.lax.broadcasted_iota(jnp.int32, ax0.shape, 0) * LANE
             + jax.lax.broadcasted_iota(jnp.int32, ax0.shape, 1)
             + j * (RB * LANE))
    validf = (a_idx < n_anchors).astype(jnp.float32)

    posf = (best_iou >= 0.5).astype(jnp.float32) * validf
    ignf = (jnp.logical_and(best_iou >= 0.4, best_iou < 0.5)
            .astype(jnp.float32) * validf)
    posf_ref[0] = posf
    ignf_ref[0] = ignf
    labf_ref[0] = best_lab
    npos_ref[0] += jnp.sum(posf, axis=0, keepdims=True)

    aw = ax1 - ax0
    ah = ay1 - ay0
    acx = ax0 + 0.5 * aw
    acy = ay0 + 0.5 * ah
    gw = bgx1 - bgx0
    gh = bgy1 - bgy0
    gcx = bgx0 + 0.5 * gw
    gcy = bgy0 + 0.5 * gh
    t0 = (gcx - acx) / aw
    t1 = (gcy - acy) / ah
    t2 = jnp.log(gw / aw)
    t3 = jnp.log(gh / ah)

    sl_acc = jnp.zeros(ax0.shape, jnp.float32)
    for k, t in enumerate((t0, t1, t2, t3)):
        diff = jnp.abs(reg_ref[0, k] - t)
        sl_acc += jnp.where(diff < DIVIDE_LINE,
                            0.5 / DIVIDE_LINE * diff * diff,
                            diff - 0.5 * DIVIDE_LINE)
    rsum_ref[0] += jnp.sum(sl_acc * posf, axis=0, keepdims=True)


def _dense_kernel(cls_ref, labi_ref, posf_ref, ignf_ref,
                  bsum_ref, corr_ref, pam_ref, *, nb):
    j = pl.program_id(0)

    @pl.when(j % nb == 0)
    def _init():
        bsum_ref[...] = jnp.zeros_like(bsum_ref)
        pam_ref[...] = jnp.zeros_like(pam_ref)

    p = jnp.clip(cls_ref[...], 1e-06, 1.0 - 1e-06)
    bsum_ref[0] += jnp.full((1, LANE), jnp.sum(jnp.log(1.0 - p) * p * p))

    labm = labi_ref[0] - (j % nb) * CR
    at_m = labm == jax.lax.broadcasted_iota(jnp.int32, p.shape, 0)
    pam_ref[...] += jnp.sum(jnp.where(at_m, p, 0.0), axis=0, keepdims=True)

    @pl.when(j % nb == nb - 1)
    def _corr():
        pm = pam_ref[...]
        qm = 1.0 - pm
        bg_m = (1.0 - ALPHA) * -jnp.log(qm) * pm * pm
        fg_m = ALPHA * -jnp.log(pm) * qm * qm
        corr = (posf_ref[0] * (fg_m - bg_m) - ignf_ref[0] * bg_m)
        corr_ref[0] = jnp.full((1, LANE), jnp.sum(corr))


@jax.jit
def kernel(classifications, regressions, anchors, gt_boxes, gt_labels):
    n, a, c = classifications.shape
    m = gt_boxes.shape[1]
    a_pad = ROWS * LANE

    anc_pad = jnp.concatenate(
        [anchors,
         jnp.broadcast_to(jnp.array([0.0, 0.0, 1.0, 1.0], jnp.float32),
                          (a_pad - a, 4))], axis=0)
    anc_t = anc_pad.T.reshape(4, ROWS, LANE)
    reg_pad = jnp.pad(regressions, ((0, 0), (0, a_pad - a), (0, 0)))
    reg_t = reg_pad.transpose(0, 2, 1).reshape(n, 4, ROWS, LANE)
    boxes_s = gt_boxes.reshape(n, 1, 4 * m)
    labels_s = gt_labels.astype(jnp.float32).reshape(n, 1, m)

    posf, ignf, labf, rsum, npos = pl.pallas_call(
        functools.partial(_match_kernel, n_anchors=a, n_boxes=m),
        grid=(n, ROWS // RB),
        in_specs=[
            pl.BlockSpec((1, 1, 4 * m), lambda i, j: (i, 0, 0),
                         memory_space=pltpu.MemorySpace.SMEM),
            pl.BlockSpec((1, 1, m), lambda i, j: (i, 0, 0),
                         memory_space=pltpu.MemorySpace.SMEM),
            pl.BlockSpec((4, RB, LANE), lambda i, j: (0, j, 0)),
            pl.BlockSpec((1, 4, RB, LANE), lambda i, j: (i, 0, j, 0)),
        ],
        out_specs=[
            pl.BlockSpec((1, RB, LANE), lambda i, j: (i, j, 0)),
            pl.BlockSpec((1, RB, LANE), lambda i, j: (i, j, 0)),
            pl.BlockSpec((1, RB, LANE), lambda i, j: (i, j, 0)),
            pl.BlockSpec((1, 1, LANE), lambda i, j: (i, 0, 0)),
            pl.BlockSpec((1, 1, LANE), lambda i, j: (i, 0, 0)),
        ],
        out_shape=[
            jax.ShapeDtypeStruct((n, ROWS, LANE), jnp.float32),
            jax.ShapeDtypeStruct((n, ROWS, LANE), jnp.float32),
            jax.ShapeDtypeStruct((n, ROWS, LANE), jnp.float32),
            jax.ShapeDtypeStruct((n, 1, LANE), jnp.float32),
            jax.ShapeDtypeStruct((n, 1, LANE), jnp.float32),
        ],
    )(boxes_s, labels_s, anc_t, reg_t)

    labv = labf.reshape(n, a_pad)[:, :a].reshape(n, 1, a).astype(jnp.int32)
    posv = posf.reshape(n, a_pad)[:, :a].reshape(n, 1, a)
    ignv = ignf.reshape(n, a_pad)[:, :a].reshape(n, 1, a)

    cls2d = classifications.transpose(0, 2, 1).reshape(n * c, a)
    nb = c // CR
    bsum_d, corr_d = pl.pallas_call(
        functools.partial(_dense_kernel, nb=nb),
        grid=(n * nb,),
        in_specs=[
            pl.BlockSpec((CR, a), lambda j: (j, 0)),
            pl.BlockSpec((1, 1, a), lambda j: (j // nb, 0, 0)),
            pl.BlockSpec((1, 1, a), lambda j: (j // nb, 0, 0)),
            pl.BlockSpec((1, 1, a), lambda j: (j // nb, 0, 0)),
        ],
        out_specs=[
            pl.BlockSpec((1, 1, LANE), lambda j: (j // nb, 0, 0)),
            pl.BlockSpec((1, 1, LANE), lambda j: (j // nb, 0, 0)),
        ],
        out_shape=[
            jax.ShapeDtypeStruct((n, 1, LANE), jnp.float32),
            jax.ShapeDtypeStruct((n, 1, LANE), jnp.float32),
        ],
        scratch_shapes=[pltpu.VMEM((1, a), jnp.float32)],
    )(cls2d, labv, posv, ignv)

    csum = -(1.0 - ALPHA) * bsum_d[:, 0, 0] + corr_d[:, 0, 0]
    np_ = jnp.sum(npos, axis=(1, 2))
    rs = jnp.sum(rsum, axis=(1, 2))
    denom = jnp.maximum(np_, 1.0)
    class_loss = jnp.mean(csum / denom)
    reg_loss = jnp.mean(jnp.where(np_ > 0, rs / (denom * 4.0), 0.0))
    return (class_loss, reg_loss)
